# baseline (speedup 1.0000x reference)
import jax
import jax.numpy as jnp
from jax.experimental import pallas as pl
from jax.experimental.pallas import tpu as pltpu

_NUM_TYPE = 1000
_BATCH = 1024
_HIST = 200
_TN = 200


def _body(ti_ref, tjs_ref, h_ref, emb_ref, out_ref, gtT_ref, hf_ref):
    @pl.when(pl.program_id(0) == 0)
    def _init():
        gtT_ref[:] = jnp.exp(tjs_ref[0, :][:, None] - ti_ref[:, 0][None, :])
        hf_ref[:] = h_ref[0].astype(jnp.float32)

    P = jax.lax.dot_general(
        emb_ref[:], hf_ref[:], (((1,), (1,)), ((), ())),
        preferred_element_type=jnp.float32)
    out_ref[:] = jnp.dot(P, gtT_ref[:], preferred_element_type=jnp.float32)


def kernel(ti, tjs, ci, cjs, emb_weight):
    del ci
    grid = (_NUM_TYPE // _TN,)
    out = pl.pallas_call(
        _body,
        grid=grid,
        in_specs=[
            pl.BlockSpec((_BATCH, 1), lambda i: (0, 0)),
            pl.BlockSpec((1, _HIST), lambda i: (0, 0)),
            pl.BlockSpec((1, _HIST, _NUM_TYPE), lambda i: (0, 0, 0)),
            pl.BlockSpec((_TN, _NUM_TYPE), lambda i: (i, 0)),
        ],
        out_specs=pl.BlockSpec((_TN, _BATCH), lambda i: (i, 0)),
        out_shape=jax.ShapeDtypeStruct((_NUM_TYPE, _BATCH), jnp.float32),
        scratch_shapes=[
            pltpu.VMEM((_HIST, _BATCH), jnp.float32),
            pltpu.VMEM((_HIST, _NUM_TYPE), jnp.float32),
        ],
    )(ti, tjs, cjs, emb_weight)
    return out[:, :, None]


# speedup vs baseline: 2.2893x; 2.2893x over previous
import jax
import jax.numpy as jnp
from jax.experimental import pallas as pl
from jax.experimental.pallas import tpu as pltpu

_NUM_TYPE = 1000
_BATCH = 1024
_HIST = 200
_TN = 200
_LANES = 128
_SUB = _BATCH // _LANES


def _body(ti_ref, tjs_ref, h_ref, emb_ref, out_ref, gtT_ref, hf_ref):
    @pl.when(pl.program_id(0) == 0)
    def _init():
        gtT_ref[:] = jnp.exp(tjs_ref[0, :][:, None] - ti_ref[0, :][None, :])
        hf_ref[:] = h_ref[0].astype(jnp.float32)

    P = jax.lax.dot_general(
        emb_ref[:], hf_ref[:], (((1,), (1,)), ((), ())),
        preferred_element_type=jnp.float32)
    res = jnp.dot(P, gtT_ref[:], preferred_element_type=jnp.float32)
    out_ref[:] = res.reshape(_TN, _SUB, _LANES)


def kernel(ti, tjs, ci, cjs, emb_weight):
    del ci
    ti_row = jnp.reshape(ti, (1, _BATCH))
    grid = (_NUM_TYPE // _TN,)
    out = pl.pallas_call(
        _body,
        grid=grid,
        in_specs=[
            pl.BlockSpec((1, _BATCH), lambda i: (0, 0)),
            pl.BlockSpec((1, _HIST), lambda i: (0, 0)),
            pl.BlockSpec((1, _HIST, _NUM_TYPE), lambda i: (0, 0, 0)),
            pl.BlockSpec((_TN, _NUM_TYPE), lambda i: (i, 0)),
        ],
        out_specs=pl.BlockSpec((_TN, _SUB, _LANES), lambda i: (i, 0, 0)),
        out_shape=jax.ShapeDtypeStruct((_NUM_TYPE, _SUB, _LANES), jnp.float32),
        scratch_shapes=[
            pltpu.VMEM((_HIST, _BATCH), jnp.float32),
            pltpu.VMEM((_HIST, _NUM_TYPE), jnp.float32),
        ],
    )(ti_row, tjs, cjs, emb_weight)
    return jnp.reshape(out, (_NUM_TYPE, _BATCH, 1))
